# baseline (speedup 1.0000x reference)
import functools

import jax
import jax.numpy as jnp
from jax.experimental import pallas as pl
from jax.experimental.pallas import tpu as pltpu
from jax.experimental.pallas import tpu_sc as plsc

_NUM_CORES = 2
_NUM_SUBCORES = 16
_NUM_WORKERS = _NUM_CORES * _NUM_SUBCORES


def _emb_gather_sc(table, idx):
    n = idx.shape[0]
    d = table.shape[1]
    per_w = n // _NUM_WORKERS
    n_chunks = 4
    rows_c = per_w // n_chunks
    mesh = plsc.VectorSubcoreMesh(
        core_axis_name="c", subcore_axis_name="s",
        num_cores=_NUM_CORES, num_subcores=_NUM_SUBCORES)

    @functools.partial(
        pl.kernel,
        mesh=mesh,
        out_type=jax.ShapeDtypeStruct((n, d), table.dtype),
        scratch_types=[
            pltpu.VMEM((per_w,), jnp.int32),
            pltpu.VMEM((rows_c, d), table.dtype),
            pltpu.VMEM((rows_c, d), table.dtype),
            pltpu.SemaphoreType.DMA,
            pltpu.SemaphoreType.DMA,
            pltpu.SemaphoreType.DMA,
        ],
    )
    def gather_kernel(table_hbm, idx_hbm, out_hbm, idx_v, buf0, buf1,
                      gsem, ssem0, ssem1):
        wid = jax.lax.axis_index("s") * _NUM_CORES + jax.lax.axis_index("c")
        base = wid * per_w
        pltpu.sync_copy(idx_hbm.at[pl.ds(base, per_w)], idx_v)
        bufs = (buf0, buf1)
        ssems = (ssem0, ssem1)
        scats = [None, None]
        for c in range(n_chunks):
            b = c % 2
            if scats[b] is not None:
                scats[b].wait()
            pltpu.async_copy(
                table_hbm.at[idx_v.at[pl.ds(c * rows_c, rows_c)]],
                bufs[b], gsem).wait()
            scats[b] = pltpu.async_copy(
                bufs[b], out_hbm.at[pl.ds(base + c * rows_c, rows_c)],
                ssems[b])
        scats[0].wait()
        scats[1].wait()

    return gather_kernel(table, idx)


def _recurrence(emb_all, meta, lens2, worlds, U, V, interpret=False):
    L, B, D = emb_all.shape
    H = D // 2

    def body(emb_ref, meta_ref, lens_ref, worlds_ref, U_ref, V_ref,
             out_ref, S_ref, cib_ref):
        wmean = jnp.mean(worlds_ref[...], axis=0, keepdims=True)
        ctx = jnp.dot(wmean, V_ref[...],
                      preferred_element_type=jnp.float32)
        S_ref[0] = jnp.zeros((B, D), jnp.float32)
        lens = jnp.maximum(lens_ref[...], 1)
        Um = U_ref[...]
        col = jax.lax.broadcasted_iota(jnp.int32, (3, 3 * D), 1) // D
        row = jax.lax.broadcasted_iota(jnp.int32, (3, 3 * D), 0)
        Wb = (col == row).astype(jnp.float32)
        zh = jnp.zeros((B, H), jnp.float32)

        def mk_cib(s):
            nf = jnp.asarray(s + 1, jnp.float32)
            af = meta_ref[s].astype(jnp.float32)
            q = jnp.floor(af / nf + 1e-3)
            cif = af - nf * q
            ci0 = cif[:, 0:1]
            ci1 = cif[:, 1:2]
            dup = ci0 == ci1
            ci1p = jnp.where(dup, -1.0, ci1)
            dsc = jnp.where(dup, 2.0, 1.0)
            lhs = jnp.concatenate([ci0, ci1p, dsc], axis=1)
            return jnp.dot(lhs, Wb, preferred_element_type=jnp.float32)

        cib_ref[0] = mk_cib(0)

        def mk_blk(masks):
            c0L, c0H, c1L, c1H = masks

            def blk(j, carry):
                aL, aH = carry
                for t in range(8):
                    k = j * 8 + t
                    kf = k.astype(jnp.float32)
                    s_k = S_ref[k]
                    aL = aL + jnp.where((c0L == kf) | (c1L == kf),
                                        s_k[:, :H], 0.0)
                    aH = aH + jnp.where((c0H == kf) | (c1H == kf),
                                        s_k[:, H:], 0.0)
                return aL, aH

            return blk

        def step(s, carry):
            out_acc, pL, pH = carry
            cur = s & 1
            tmasks = (cib_ref[cur, :, 0:H], cib_ref[cur, :, H:D],
                      cib_ref[cur, :, D:D + H], cib_ref[cur, :, D + H:2 * D])
            tL, tH = mk_blk(tmasks)(s // 8, (zh, zh))
            lr = (jnp.concatenate([pL + tL, pH + tH], axis=1)
                  * cib_ref[cur, :, 2 * D:])
            pre = emb_ref[s] + jnp.dot(lr, Um,
                                       preferred_element_type=jnp.float32)
            cibn = mk_cib(jnp.minimum(s + 1, L - 1))
            nmasks = (cibn[:, 0:H], cibn[:, H:D],
                      cibn[:, D:D + H], cibn[:, D + H:2 * D])
            npre = jnp.where(s < L - 1, (s + 1) // 8, 0)
            nL, nH = jax.lax.fori_loop(0, npre, mk_blk(nmasks), (zh, zh))
            h = jnp.where(s < lens, jnp.tanh(pre + ctx), 0.0)
            S_ref[s + 1] = h
            cib_ref[1 - cur] = cibn
            out_acc = jnp.where(lens == s + 1, h, out_acc)
            return out_acc, nL, nH

        out_acc, _, _ = jax.lax.fori_loop(
            0, L, step, (jnp.zeros((B, D), jnp.float32), zh, zh))
        out_ref[...] = out_acc

    return pl.pallas_call(
        body,
        out_shape=jax.ShapeDtypeStruct((B, D), jnp.float32),
        scratch_shapes=[pltpu.VMEM((L + 1, B, D), jnp.float32),
                        pltpu.VMEM((2, B, 3 * D), jnp.float32)],
        interpret=interpret,
    )(emb_all, meta, lens2, worlds, U, V)


def kernel(worlds, symbols, args, lengths, E_sym, U, V):
    B, L = symbols.shape
    D = E_sym.shape[1]
    sym_flat = symbols.T.reshape(-1)
    emb_all = _emb_gather_sc(E_sym, sym_flat).reshape(L, B, D)
    meta = args.transpose(1, 0, 2)
    lens2 = lengths.reshape(B, 1)
    return _recurrence(emb_all, meta, lens2, worlds, U, V)


# speedup vs baseline: 1.0005x; 1.0005x over previous
import functools

import jax
import jax.numpy as jnp
from jax.experimental import pallas as pl
from jax.experimental.pallas import tpu as pltpu
from jax.experimental.pallas import tpu_sc as plsc

_NUM_CORES = 2
_NUM_SUBCORES = 16
_NUM_WORKERS = _NUM_CORES * _NUM_SUBCORES


def _emb_gather_sc(table, idx, Bt, Lt):
    n = idx.shape[0]
    d = table.shape[1]
    per_w = n // _NUM_WORKERS
    trees_w = per_w // Lt
    n_chunks = 4
    rows_c = per_w // n_chunks
    mesh = plsc.VectorSubcoreMesh(
        core_axis_name="c", subcore_axis_name="s",
        num_cores=_NUM_CORES, num_subcores=_NUM_SUBCORES)

    @functools.partial(
        pl.kernel,
        mesh=mesh,
        out_type=jax.ShapeDtypeStruct((n, d), table.dtype),
        scratch_types=[
            pltpu.VMEM((per_w,), jnp.int32),
            pltpu.VMEM((n_chunks, rows_c), jnp.int32),
            pltpu.VMEM((rows_c, d), table.dtype),
            pltpu.VMEM((rows_c, d), table.dtype),
            pltpu.SemaphoreType.DMA,
            pltpu.SemaphoreType.DMA,
            pltpu.SemaphoreType.DMA,
        ],
    )
    def gather_kernel(table_hbm, idx_hbm, out_hbm, idx_v, oidx_v, buf0, buf1,
                      gsem, ssem0, ssem1):
        wid = jax.lax.axis_index("s") * _NUM_CORES + jax.lax.axis_index("c")
        base = wid * per_w
        pltpu.sync_copy(idx_hbm.at[pl.ds(base, per_w)], idx_v)
        lane = jax.lax.iota(jnp.int32, 16)
        for c in range(n_chunks):
            for j in range(rows_c // 16):
                i0 = c * rows_c + j * 16
                orow = (lane + (i0 % Lt)) * Bt + wid * trees_w + i0 // Lt
                oidx_v[c, pl.ds(j * 16, 16)] = orow
        bufs = (buf0, buf1)
        ssems = (ssem0, ssem1)
        scats = [None, None]
        for c in range(n_chunks):
            b = c % 2
            if scats[b] is not None:
                scats[b].wait()
            pltpu.async_copy(
                table_hbm.at[idx_v.at[pl.ds(c * rows_c, rows_c)]],
                bufs[b], gsem).wait()
            scats[b] = pltpu.async_copy(
                bufs[b], out_hbm.at[oidx_v.at[c]], ssems[b])
        scats[0].wait()
        scats[1].wait()

    return gather_kernel(table, idx)


def _recurrence(emb_all, meta, lens2, worlds, U, V, interpret=False):
    L, B, D = emb_all.shape
    H = D // 2

    def body(emb_ref, meta_ref, lens_ref, worlds_ref, U_ref, V_ref,
             out_ref, S_ref, cib_ref):
        wmean = jnp.mean(worlds_ref[...], axis=0, keepdims=True)
        ctx = jnp.dot(wmean, V_ref[...],
                      preferred_element_type=jnp.float32)
        S_ref[0] = jnp.zeros((B, D), jnp.float32)
        lens = jnp.maximum(lens_ref[...], 1)
        Um = U_ref[...]
        col = jax.lax.broadcasted_iota(jnp.int32, (3, 3 * D), 1) // D
        row = jax.lax.broadcasted_iota(jnp.int32, (3, 3 * D), 0)
        Wb = (col == row).astype(jnp.float32)
        zh = jnp.zeros((B, H), jnp.float32)

        def mk_cib(s):
            nf = jnp.asarray(s + 1, jnp.float32)
            af = meta_ref[s].astype(jnp.float32)
            q = jnp.floor(af / nf + 1e-3)
            cif = af - nf * q
            ci0 = cif[:, 0:1]
            ci1 = cif[:, 1:2]
            dup = ci0 == ci1
            ci1p = jnp.where(dup, -1.0, ci1)
            dsc = jnp.where(dup, 2.0, 1.0)
            lhs = jnp.concatenate([ci0, ci1p, dsc], axis=1)
            return jnp.dot(lhs, Wb, preferred_element_type=jnp.float32)

        cib_ref[0] = mk_cib(0)

        def step(s, out_acc):
            cur = s & 1
            c0L = cib_ref[cur, :, 0:H]
            c0H = cib_ref[cur, :, H:D]
            c1L = cib_ref[cur, :, D:D + H]
            c1H = cib_ref[cur, :, D + H:2 * D]

            def blk(j, carry):
                aL, aH = carry
                for t in range(8):
                    k = j * 8 + t
                    kf = k.astype(jnp.float32)
                    s_k = S_ref[k]
                    aL = aL + jnp.where((c0L == kf) | (c1L == kf),
                                        s_k[:, :H], 0.0)
                    aH = aH + jnp.where((c0H == kf) | (c1H == kf),
                                        s_k[:, H:], 0.0)
                return aL, aH

            aL, aH = jax.lax.fori_loop(0, s // 8 + 1, blk, (zh, zh))
            lr = (jnp.concatenate([aL, aH], axis=1)
                  * cib_ref[cur, :, 2 * D:])
            pre = emb_ref[s] + jnp.dot(lr, Um,
                                       preferred_element_type=jnp.float32)
            h = jnp.where(s < lens, jnp.tanh(pre + ctx), 0.0)
            S_ref[s + 1] = h
            cib_ref[1 - cur] = mk_cib(jnp.minimum(s + 1, L - 1))
            return jnp.where(lens == s + 1, h, out_acc)

        out_ref[...] = jax.lax.fori_loop(0, L, step,
                                         jnp.zeros((B, D), jnp.float32))

    return pl.pallas_call(
        body,
        out_shape=jax.ShapeDtypeStruct((B, D), jnp.float32),
        scratch_shapes=[pltpu.VMEM((L + 1, B, D), jnp.float32),
                        pltpu.VMEM((2, B, 3 * D), jnp.float32)],
        interpret=interpret,
    )(emb_all, meta, lens2, worlds, U, V)


def kernel(worlds, symbols, args, lengths, E_sym, U, V):
    B, L = symbols.shape
    D = E_sym.shape[1]
    sym_flat = symbols.reshape(-1)
    emb_all = _emb_gather_sc(E_sym, sym_flat, B, L).reshape(L, B, D)
    meta = args.transpose(1, 0, 2)
    lens2 = lengths.reshape(B, 1)
    return _recurrence(emb_all, meta, lens2, worlds, U, V)


# speedup vs baseline: 1.0088x; 1.0083x over previous
import functools

import jax
import jax.numpy as jnp
from jax.experimental import pallas as pl
from jax.experimental.pallas import tpu as pltpu
from jax.experimental.pallas import tpu_sc as plsc

_NUM_CORES = 2
_NUM_SUBCORES = 16
_NUM_WORKERS = _NUM_CORES * _NUM_SUBCORES


def _emb_gather_sc(table, idx, Bt, Lt):
    n = idx.shape[0]
    d = table.shape[1]
    per_w = n // _NUM_WORKERS
    trees_w = per_w // Lt
    n_chunks = 4
    rows_c = per_w // n_chunks
    mesh = plsc.VectorSubcoreMesh(
        core_axis_name="c", subcore_axis_name="s",
        num_cores=_NUM_CORES, num_subcores=_NUM_SUBCORES)

    @functools.partial(
        pl.kernel,
        mesh=mesh,
        out_type=jax.ShapeDtypeStruct((n, d), table.dtype),
        scratch_types=[
            pltpu.VMEM((per_w,), jnp.int32),
            pltpu.VMEM((n_chunks, rows_c), jnp.int32),
            pltpu.VMEM((rows_c, d), table.dtype),
            pltpu.VMEM((rows_c, d), table.dtype),
            pltpu.SemaphoreType.DMA,
            pltpu.SemaphoreType.DMA,
            pltpu.SemaphoreType.DMA,
        ],
    )
    def gather_kernel(table_hbm, idx_hbm, out_hbm, idx_v, oidx_v, buf0, buf1,
                      gsem, ssem0, ssem1):
        wid = jax.lax.axis_index("s") * _NUM_CORES + jax.lax.axis_index("c")
        base = wid * per_w
        pltpu.sync_copy(idx_hbm.at[pl.ds(base, per_w)], idx_v)
        lane = jax.lax.iota(jnp.int32, 16)
        for c in range(n_chunks):
            for j in range(rows_c // 16):
                i0 = c * rows_c + j * 16
                orow = (lane + (i0 % Lt)) * Bt + wid * trees_w + i0 // Lt
                oidx_v[c, pl.ds(j * 16, 16)] = orow
        bufs = (buf0, buf1)
        ssems = (ssem0, ssem1)
        scats = [None, None]
        for c in range(n_chunks):
            b = c % 2
            if scats[b] is not None:
                scats[b].wait()
            pltpu.async_copy(
                table_hbm.at[idx_v.at[pl.ds(c * rows_c, rows_c)]],
                bufs[b], gsem).wait()
            scats[b] = pltpu.async_copy(
                bufs[b], out_hbm.at[oidx_v.at[c]], ssems[b])
        scats[0].wait()
        scats[1].wait()

    return gather_kernel(table, idx)


def _recurrence(emb_all, meta, lens2, worlds, U, V, interpret=False):
    L, B, D = emb_all.shape
    H = D // 2

    def body(emb_ref, meta_ref, lens_ref, worlds_ref, U_ref, V_ref,
             out_ref, S_ref, cib_ref):
        wmean = jnp.mean(worlds_ref[...], axis=0, keepdims=True)
        ctx = jnp.dot(wmean, V_ref[...],
                      preferred_element_type=jnp.float32)
        S_ref[0] = jnp.zeros((B, D), jnp.float32)
        lens = jnp.maximum(lens_ref[...], 1)
        Um = U_ref[...]
        col = jax.lax.broadcasted_iota(jnp.int32, (3, 3 * D), 1) // D
        row = jax.lax.broadcasted_iota(jnp.int32, (3, 3 * D), 0)
        Wb = (col == row).astype(jnp.float32)
        zh = jnp.zeros((B, H), jnp.float32)

        def mk_cib(s):
            nf = jnp.asarray(s + 1, jnp.float32)
            af = meta_ref[s].astype(jnp.float32)
            q = jnp.floor(af / nf + 1e-3)
            cif = af - nf * q
            ci0 = cif[:, 0:1]
            ci1 = cif[:, 1:2]
            dup = ci0 == ci1
            ci1p = jnp.where(dup, -1.0, ci1)
            dsc = jnp.where(dup, 2.0, 1.0)
            lhs = jnp.concatenate([ci0, ci1p, dsc], axis=1)
            return jnp.dot(lhs, Wb, preferred_element_type=jnp.float32)

        cib_ref[0] = mk_cib(0)

        def one_step(s, cur, nxt, out_acc):
            c0L = cib_ref[cur, :, 0:H]
            c0H = cib_ref[cur, :, H:D]
            c1L = cib_ref[cur, :, D:D + H]
            c1H = cib_ref[cur, :, D + H:2 * D]

            def blk(j, carry):
                aL, aH = carry
                for t in range(8):
                    k = j * 8 + t
                    kf = k.astype(jnp.float32)
                    s_k = S_ref[k]
                    aL = aL + jnp.where((c0L == kf) | (c1L == kf),
                                        s_k[:, :H], 0.0)
                    aH = aH + jnp.where((c0H == kf) | (c1H == kf),
                                        s_k[:, H:], 0.0)
                return aL, aH

            aL, aH = jax.lax.fori_loop(0, s // 8 + 1, blk, (zh, zh))
            lr = (jnp.concatenate([aL, aH], axis=1)
                  * cib_ref[cur, :, 2 * D:])
            pre = emb_ref[s] + jnp.dot(lr, Um,
                                       preferred_element_type=jnp.float32)
            h = jnp.where(s < lens, jnp.tanh(pre + ctx), 0.0)
            S_ref[s + 1] = h
            cib_ref[nxt] = mk_cib(jnp.minimum(s + 1, L - 1))
            return jnp.where(lens == s + 1, h, out_acc)

        def pair(i, out_acc):
            out_acc = one_step(2 * i, 0, 1, out_acc)
            return one_step(2 * i + 1, 1, 0, out_acc)

        out_ref[...] = jax.lax.fori_loop(0, L // 2, pair,
                                         jnp.zeros((B, D), jnp.float32))

    return pl.pallas_call(
        body,
        out_shape=jax.ShapeDtypeStruct((B, D), jnp.float32),
        scratch_shapes=[pltpu.VMEM((L + 1, B, D), jnp.float32),
                        pltpu.VMEM((2, B, 3 * D), jnp.float32)],
        interpret=interpret,
    )(emb_all, meta, lens2, worlds, U, V)


def kernel(worlds, symbols, args, lengths, E_sym, U, V):
    B, L = symbols.shape
    D = E_sym.shape[1]
    sym_flat = symbols.reshape(-1)
    emb_all = _emb_gather_sc(E_sym, sym_flat, B, L).reshape(L, B, D)
    meta = args.transpose(1, 0, 2)
    lens2 = lengths.reshape(B, 1)
    return _recurrence(emb_all, meta, lens2, worlds, U, V)
